# baseline (speedup 1.0000x reference)
import functools

import jax
import jax.numpy as jnp
from jax import lax
from jax.experimental import pallas as pl
from jax.experimental.pallas import tpu as pltpu
from jax.experimental.pallas import tpu_sc as plsc

P, K, D = 1024, 8192, 256
KB = 1024
NUM_BLOCKS = K // KB

_NC, _NS = 2, 16
_NW = _NC * _NS
_BPW = P // _NW


def _argmin_body(a_ref, c_ref, ids_ref, bv_ref, bi_ref):
    i = pl.program_id(0)
    a = a_ref[...]
    c = c_ref[...]
    a2 = jnp.sum(a * a, axis=1, keepdims=True)
    b2 = jnp.sum(c * c, axis=1)
    ab = lax.dot_general(a, c, (((1,), (1,)), ((), ())),
                         preferred_element_type=jnp.float32)
    d2 = a2 + b2[None, :] - 2.0 * ab
    d = jnp.sqrt(jnp.maximum(d2, 0.0))
    vmin = jnp.min(d, axis=1, keepdims=True)
    iota = lax.broadcasted_iota(jnp.int32, d.shape, 1)
    imin = jnp.min(jnp.where(d == vmin, iota, jnp.int32(KB)),
                   axis=1, keepdims=True) + i * KB

    @pl.when(i == 0)
    def _():
        bv_ref[...] = vmin
        bi_ref[...] = imin

    @pl.when(i > 0)
    def _():
        better = vmin < bv_ref[...]
        bv_ref[...] = jnp.where(better, vmin, bv_ref[...])
        bi_ref[...] = jnp.where(better, imin, bi_ref[...])

    @pl.when(i == NUM_BLOCKS - 1)
    def _():
        ids_ref[...] = bi_ref[...]


def _argmin_ids(prompt_embs, clip_embs):
    ids2d = pl.pallas_call(
        _argmin_body,
        grid=(NUM_BLOCKS,),
        in_specs=[
            pl.BlockSpec((P, D), lambda i: (0, 0)),
            pl.BlockSpec((KB, D), lambda i: (i, 0)),
        ],
        out_specs=pl.BlockSpec((P, 1), lambda i: (0, 0)),
        out_shape=jax.ShapeDtypeStruct((P, 1), jnp.int32),
        scratch_shapes=[
            pltpu.VMEM((P, 1), jnp.float32),
            pltpu.VMEM((P, 1), jnp.int32),
        ],
    )(prompt_embs, clip_embs)
    return ids2d.reshape(P)


@functools.partial(
    pl.kernel,
    mesh=plsc.VectorSubcoreMesh(core_axis_name="c", subcore_axis_name="s"),
    out_type=jax.ShapeDtypeStruct((P, D), jnp.float32),
    scratch_types=[
        pltpu.VMEM((_BPW,), jnp.int32),
        pltpu.VMEM((_BPW, D), jnp.float32),
        pltpu.SemaphoreType.DMA,
    ],
)
def _sc_gather(table_hbm, idx_hbm, out_hbm, idx_v, rows_v, sem):
    wid = lax.axis_index("s") * _NC + lax.axis_index("c")
    base = wid * _BPW
    pltpu.sync_copy(idx_hbm.at[pl.ds(base, _BPW)], idx_v)
    pltpu.async_copy(table_hbm.at[idx_v], rows_v, sem).wait()
    pltpu.sync_copy(rows_v, out_hbm.at[pl.ds(base, _BPW)])


def kernel(prompt_embs, clip_embs):
    ids = _argmin_ids(prompt_embs, clip_embs)
    vocab = _sc_gather(clip_embs, ids)
    out = lax.stop_gradient(vocab - prompt_embs) + prompt_embs
    return (out, ids)


# speedup vs baseline: 2.4144x; 2.4144x over previous
import jax
import jax.numpy as jnp
from jax import lax
from jax.experimental import pallas as pl
from jax.experimental.pallas import tpu as pltpu

P, K, D = 1024, 8192, 256
KB = 1024
NUM_BLOCKS = K // KB


def _body(a_ref, c_ref, out_ref, ids_ref, bv_ref, bi_ref, br_ref):
    i = pl.program_id(0)
    a = a_ref[...]
    c = c_ref[...]
    a2 = jnp.sum(a * a, axis=1, keepdims=True)
    b2 = jnp.sum(c * c, axis=1)
    m2ab = lax.dot_general(-2.0 * a, c, (((1,), (1,)), ((), ())),
                           preferred_element_type=jnp.float32)
    d2 = (a2 + b2[None, :]) + m2ab
    d = jnp.sqrt(jnp.maximum(d2, 0.0))
    vmin = jnp.min(d, axis=1, keepdims=True)
    iota = lax.broadcasted_iota(jnp.int32, d.shape, 1)
    imin = jnp.min(jnp.where(d == vmin, iota, jnp.int32(KB)),
                   axis=1, keepdims=True)
    onehot = jnp.where(iota == imin, 1.0, 0.0)
    row = lax.dot_general(onehot, c, (((1,), (0,)), ((), ())),
                          preferred_element_type=jnp.float32)
    gidx = imin + i * KB

    @pl.when(i == 0)
    def _():
        bv_ref[...] = vmin
        bi_ref[...] = gidx
        br_ref[...] = row

    @pl.when(i > 0)
    def _():
        better = vmin < bv_ref[...]
        bv_ref[...] = jnp.where(better, vmin, bv_ref[...])
        bi_ref[...] = jnp.where(better, gidx, bi_ref[...])
        br_ref[...] = jnp.where(better, row, br_ref[...])

    @pl.when(i == NUM_BLOCKS - 1)
    def _():
        ids_ref[...] = bi_ref[...]
        out_ref[...] = (br_ref[...] - a) + a


def kernel(prompt_embs, clip_embs):
    out, ids2d = pl.pallas_call(
        _body,
        grid=(NUM_BLOCKS,),
        in_specs=[
            pl.BlockSpec((P, D), lambda i: (0, 0)),
            pl.BlockSpec((KB, D), lambda i: (i, 0)),
        ],
        out_specs=[
            pl.BlockSpec((P, D), lambda i: (0, 0)),
            pl.BlockSpec((P, 1), lambda i: (0, 0)),
        ],
        out_shape=[
            jax.ShapeDtypeStruct((P, D), jnp.float32),
            jax.ShapeDtypeStruct((P, 1), jnp.int32),
        ],
        scratch_shapes=[
            pltpu.VMEM((P, 1), jnp.float32),
            pltpu.VMEM((P, 1), jnp.int32),
            pltpu.VMEM((P, D), jnp.float32),
        ],
    )(prompt_embs, clip_embs)
    return (out, ids2d.reshape(P))


# speedup vs baseline: 2.5612x; 1.0608x over previous
import jax
import jax.numpy as jnp
from jax import lax
from jax.experimental import pallas as pl
from jax.experimental.pallas import tpu as pltpu

P, K, D = 1024, 8192, 256
KB = 4096
NUM_BLOCKS = K // KB


def _body(a_ref, c_ref, out_ref, ids_ref, bv_ref, bi_ref, br_ref):
    i = pl.program_id(0)
    a = a_ref[...]
    c = c_ref[...]
    a2 = jnp.sum(a * a, axis=1, keepdims=True)
    b2 = jnp.sum(c * c, axis=1)
    m2ab = lax.dot_general(-2.0 * a, c, (((1,), (1,)), ((), ())),
                           preferred_element_type=jnp.float32)
    d2 = (a2 + b2[None, :]) + m2ab
    d = jnp.sqrt(jnp.maximum(d2, 0.0))
    vmin = jnp.min(d, axis=1, keepdims=True)
    iota = lax.broadcasted_iota(jnp.int32, d.shape, 1)
    imin = jnp.min(jnp.where(d == vmin, iota, jnp.int32(KB)),
                   axis=1, keepdims=True)
    onehot = jnp.where(iota == imin, 1.0, 0.0)
    row = lax.dot_general(onehot, c, (((1,), (0,)), ((), ())),
                          preferred_element_type=jnp.float32)
    gidx = imin + i * KB

    @pl.when(i == 0)
    def _():
        bv_ref[...] = vmin
        bi_ref[...] = gidx
        br_ref[...] = row

    @pl.when(i > 0)
    def _():
        better = vmin < bv_ref[...]
        bv_ref[...] = jnp.where(better, vmin, bv_ref[...])
        bi_ref[...] = jnp.where(better, gidx, bi_ref[...])
        br_ref[...] = jnp.where(better, row, br_ref[...])

    @pl.when(i == NUM_BLOCKS - 1)
    def _():
        ids_ref[...] = bi_ref[...]
        out_ref[...] = (br_ref[...] - a) + a


def kernel(prompt_embs, clip_embs):
    out, ids2d = pl.pallas_call(
        _body,
        grid=(NUM_BLOCKS,),
        in_specs=[
            pl.BlockSpec((P, D), lambda i: (0, 0)),
            pl.BlockSpec((KB, D), lambda i: (i, 0)),
        ],
        out_specs=[
            pl.BlockSpec((P, D), lambda i: (0, 0)),
            pl.BlockSpec((P, 1), lambda i: (0, 0)),
        ],
        out_shape=[
            jax.ShapeDtypeStruct((P, D), jnp.float32),
            jax.ShapeDtypeStruct((P, 1), jnp.int32),
        ],
        scratch_shapes=[
            pltpu.VMEM((P, 1), jnp.float32),
            pltpu.VMEM((P, 1), jnp.int32),
            pltpu.VMEM((P, D), jnp.float32),
        ],
    )(prompt_embs, clip_embs)
    return (out, ids2d.reshape(P))
